# baseline (speedup 1.0000x reference)
import functools

import jax
import jax.numpy as jnp
from jax import lax
from jax.experimental import pallas as pl


def _fused_body(fine_ref, coarse_t_ref, feats_ref, w1t_ref, b1_ref, w2t_ref,
                b2_ref, out_ref, *, bm, kk):
    x = fine_ref[0]
    y = coarse_t_ref[0]
    f = feats_ref[0]

    x0 = x[:, 0:1]
    x1 = x[:, 1:2]
    x2c = x[:, 2:3]
    y0 = y[0:1, :]
    y1 = y[1:2, :]
    y2c = y[2:3, :]

    xsq = x0 * x0 + x1 * x1 + x2c * x2c
    ysq = y0 * y0 + y1 * y1 + y2c * y2c
    xy = jnp.dot(x.astype(jnp.bfloat16), y.astype(jnp.bfloat16),
                 preferred_element_type=jnp.float32)
    r = xy - 0.5 * ysq

    ninf = jnp.float32(-jnp.inf)
    lanes = 128
    a1 = r[:, 0:lanes]
    a2 = jnp.full((bm, lanes), ninf, dtype=jnp.float32)
    a3 = jnp.full((bm, lanes), ninf, dtype=jnp.float32)
    for g in range(1, kk // lanes):
        vg = r[:, g * lanes:(g + 1) * lanes]
        lo = jnp.minimum(a1, vg)
        a1 = jnp.maximum(a1, vg)
        a3 = jnp.maximum(a3, jnp.minimum(a2, lo))
        a2 = jnp.maximum(a2, lo)
    v1 = jnp.max(a1, axis=1, keepdims=True)
    head2 = jnp.where(a1 == v1, a2, a1)
    v2 = jnp.max(head2, axis=1, keepdims=True)
    head3 = jnp.where(head2 == v2, jnp.where(a1 == v1, a3, a2), head2)
    v3 = jnp.max(head3, axis=1, keepdims=True)
    lt1 = r < v1
    lt2 = r < v2
    lt3 = r < v3

    def w_of(v):
        return lax.rsqrt(jnp.maximum(xsq - 2.0 * v, 1e-12))

    w1, w2, w3 = w_of(v1), w_of(v2), w_of(v3)
    wsum = w1 + w2 + w3
    zero = jnp.zeros((bm, kk), dtype=jnp.float32)
    s = jnp.where(lt1, jnp.where(lt2, jnp.where(lt3, zero, w3), w2), w1)

    interp = jnp.dot(s, f, preferred_element_type=jnp.float32)
    interp = interp * (1.0 / wsum)
    h = interp @ w1t_ref[...] + b1_ref[...]
    h = 0.5 * h * (1.0 + lax.erf(h * jnp.float32(0.7071067811865476)))
    out = h @ w2t_ref[...] + b2_ref[...]
    out_ref[0] = out


def kernel(fine_coords, coarse_coords, coarse_feats, W1, b1, W2, b2):
    B, M, _ = fine_coords.shape
    _, K, C = coarse_feats.shape
    O = W1.shape[0]
    BM = 1024

    coarse_t = coarse_coords.transpose(0, 2, 1)
    w1t = W1.T
    w2t = W2.T
    b1r = b1.reshape(1, O)
    b2r = b2.reshape(1, O)

    grid = (B, M // BM)
    body = functools.partial(_fused_body, bm=BM, kk=K)
    return pl.pallas_call(
        body,
        grid=grid,
        in_specs=[
            pl.BlockSpec((1, BM, 3), lambda b, m: (b, m, 0)),
            pl.BlockSpec((1, 3, K), lambda b, m: (b, 0, 0)),
            pl.BlockSpec((1, K, C), lambda b, m: (b, 0, 0)),
            pl.BlockSpec((C, O), lambda b, m: (0, 0)),
            pl.BlockSpec((1, O), lambda b, m: (0, 0)),
            pl.BlockSpec((O, O), lambda b, m: (0, 0)),
            pl.BlockSpec((1, O), lambda b, m: (0, 0)),
        ],
        out_specs=pl.BlockSpec((1, BM, O), lambda b, m: (b, m, 0)),
        out_shape=jax.ShapeDtypeStruct((B, M, O), jnp.float32),
    )(fine_coords, coarse_t, coarse_feats, w1t, b1r, w2t, b2r)


# speedup vs baseline: 1.3506x; 1.3506x over previous
import functools

import jax
import jax.numpy as jnp
from jax import lax
from jax.experimental import pallas as pl


def _fused_body(fine_ref, coarse_t_ref, feats_ref, w1t_ref, b1_ref, w2t_ref,
                b2_ref, out_ref, *, bm, kk):
    x = fine_ref[0]
    y = coarse_t_ref[0]
    f = feats_ref[0]

    x0 = x[:, 0:1]
    x1 = x[:, 1:2]
    x2c = x[:, 2:3]
    y0 = y[0:1, :]
    y1 = y[1:2, :]
    y2c = y[2:3, :]

    xsq = x0 * x0 + x1 * x1 + x2c * x2c
    ysq = y0 * y0 + y1 * y1 + y2c * y2c
    xy = jnp.dot(x.astype(jnp.bfloat16), y.astype(jnp.bfloat16),
                 preferred_element_type=jnp.float32)
    r = xy - 0.5 * ysq

    ninf = jnp.float32(-jnp.inf)
    v1 = jnp.max(r, axis=1, keepdims=True)
    lt1 = r < v1
    v2 = jnp.max(jnp.where(lt1, r, ninf), axis=1, keepdims=True)
    lt2 = r < v2
    v3 = jnp.max(jnp.where(lt2, r, ninf), axis=1, keepdims=True)
    lt3 = r < v3

    def w_of(v):
        return lax.rsqrt(jnp.maximum(xsq - 2.0 * v, 1e-12))

    w1, w2, w3 = w_of(v1), w_of(v2), w_of(v3)
    wsum = w1 + w2 + w3
    zero = jnp.zeros((bm, kk), dtype=jnp.float32)
    s = jnp.where(lt1, jnp.where(lt2, jnp.where(lt3, zero, w3), w2), w1)

    interp = jnp.dot(s, f, preferred_element_type=jnp.float32)
    interp = interp * (1.0 / wsum)
    h = interp @ w1t_ref[...] + b1_ref[...]
    h = 0.5 * h * (1.0 + lax.erf(h * jnp.float32(0.7071067811865476)))
    out = h @ w2t_ref[...] + b2_ref[...]
    out_ref[0] = out


def kernel(fine_coords, coarse_coords, coarse_feats, W1, b1, W2, b2):
    B, M, _ = fine_coords.shape
    _, K, C = coarse_feats.shape
    O = W1.shape[0]
    BM = 2048

    coarse_t = coarse_coords.transpose(0, 2, 1)
    w1t = W1.T
    w2t = W2.T
    b1r = b1.reshape(1, O)
    b2r = b2.reshape(1, O)

    grid = (B, M // BM)
    body = functools.partial(_fused_body, bm=BM, kk=K)
    return pl.pallas_call(
        body,
        grid=grid,
        in_specs=[
            pl.BlockSpec((1, BM, 3), lambda b, m: (b, m, 0)),
            pl.BlockSpec((1, 3, K), lambda b, m: (b, 0, 0)),
            pl.BlockSpec((1, K, C), lambda b, m: (b, 0, 0)),
            pl.BlockSpec((C, O), lambda b, m: (0, 0)),
            pl.BlockSpec((1, O), lambda b, m: (0, 0)),
            pl.BlockSpec((O, O), lambda b, m: (0, 0)),
            pl.BlockSpec((1, O), lambda b, m: (0, 0)),
        ],
        out_specs=pl.BlockSpec((1, BM, O), lambda b, m: (b, m, 0)),
        out_shape=jax.ShapeDtypeStruct((B, M, O), jnp.float32),
    )(fine_coords, coarse_t, coarse_feats, w1t, b1r, w2t, b2r)


# speedup vs baseline: 1.4934x; 1.1057x over previous
import functools

import jax
import jax.numpy as jnp
from jax import lax
from jax.experimental import pallas as pl


def _fused_body(fine_ref, coarse_t_ref, feats_ref, w1t_ref, b1_ref, w2t_ref,
                b2_ref, out_ref, *, bm, kk):
    x = fine_ref[0]
    y = coarse_t_ref[0]
    f = feats_ref[0]

    x0 = x[:, 0:1]
    x1 = x[:, 1:2]
    x2c = x[:, 2:3]
    y0 = y[0:1, :]
    y1 = y[1:2, :]
    y2c = y[2:3, :]

    xsq = x0 * x0 + x1 * x1 + x2c * x2c
    ysq = y0 * y0 + y1 * y1 + y2c * y2c
    xy = jnp.dot(x.astype(jnp.bfloat16), y.astype(jnp.bfloat16),
                 preferred_element_type=jnp.float32)
    r = xy - 0.5 * ysq

    ninf = jnp.float32(-jnp.inf)
    v1 = jnp.max(r, axis=1, keepdims=True)
    lt1 = r < v1
    v2 = jnp.max(jnp.where(lt1, r, ninf), axis=1, keepdims=True)
    lt2 = r < v2
    v3 = jnp.max(jnp.where(lt2, r, ninf), axis=1, keepdims=True)
    lt3 = r < v3

    def w_of(v):
        return lax.rsqrt(jnp.maximum(xsq - 2.0 * v, 1e-12))

    w1, w2, w3 = w_of(v1), w_of(v2), w_of(v3)
    wsum = w1 + w2 + w3
    zero = jnp.zeros((bm, kk), dtype=jnp.float32)
    s = jnp.where(lt1, jnp.where(lt2, jnp.where(lt3, zero, w3), w2), w1)

    interp = jnp.dot(s, f, preferred_element_type=jnp.float32)
    interp = interp * (1.0 / wsum)
    h = interp @ w1t_ref[...] + b1_ref[...]
    h = 0.5 * h * (1.0 + lax.erf(h * jnp.float32(0.7071067811865476)))
    out = h @ w2t_ref[...] + b2_ref[...]
    out_ref[0] = out


def kernel(fine_coords, coarse_coords, coarse_feats, W1, b1, W2, b2):
    B, M, _ = fine_coords.shape
    _, K, C = coarse_feats.shape
    O = W1.shape[0]
    BM = 4096

    coarse_t = coarse_coords.transpose(0, 2, 1)
    w1t = W1.T
    w2t = W2.T
    b1r = b1.reshape(1, O)
    b2r = b2.reshape(1, O)

    grid = (B, M // BM)
    body = functools.partial(_fused_body, bm=BM, kk=K)
    return pl.pallas_call(
        body,
        grid=grid,
        in_specs=[
            pl.BlockSpec((1, BM, 3), lambda b, m: (b, m, 0)),
            pl.BlockSpec((1, 3, K), lambda b, m: (b, 0, 0)),
            pl.BlockSpec((1, K, C), lambda b, m: (b, 0, 0)),
            pl.BlockSpec((C, O), lambda b, m: (0, 0)),
            pl.BlockSpec((1, O), lambda b, m: (0, 0)),
            pl.BlockSpec((O, O), lambda b, m: (0, 0)),
            pl.BlockSpec((1, O), lambda b, m: (0, 0)),
        ],
        out_specs=pl.BlockSpec((1, BM, O), lambda b, m: (b, m, 0)),
        out_shape=jax.ShapeDtypeStruct((B, M, O), jnp.float32),
    )(fine_coords, coarse_t, coarse_feats, w1t, b1r, w2t, b2r)
